# baseline (speedup 1.0000x reference)
import functools

import jax
import jax.numpy as jnp
from jax import lax
from jax.experimental import pallas as pl
from jax.experimental.pallas import tpu as pltpu
from jax.experimental.pallas import tpu_sc as plsc

_ROWS, _N = 128, 32768
_NC, _NS = 2, 16
_NW = _NC * _NS
_RPW = _ROWS // _NW
_CH = 2048
_NCH = _N // _CH


def kernel(x):
    mesh = plsc.VectorSubcoreMesh(
        core_axis_name="c", subcore_axis_name="s",
        num_cores=_NC, num_subcores=_NS)

    @functools.partial(
        pl.kernel,
        out_type=jax.ShapeDtypeStruct((_ROWS, _N), jnp.float32),
        mesh=mesh,
        compiler_params=pltpu.CompilerParams(needs_layout_passes=False),
        scratch_types=[
            pltpu.VMEM((_RPW, _CH), jnp.float32),
            pltpu.VMEM((_RPW, _CH), jnp.float32),
        ],
    )
    def run(x_hbm, o_hbm, ibuf, obuf):
        wid = lax.axis_index("s") * _NC + lax.axis_index("c")
        base = wid * _RPW
        last = jnp.full((16, 1), 15, jnp.int32)
        dnums = lax.GatherDimensionNumbers(
            offset_dims=(), collapsed_slice_dims=(0,), start_index_map=(0,))

        def bcast_last(s):
            return lax.gather(
                s, last, dnums, (1,),
                mode=lax.GatherScatterMode.PROMISE_IN_BOUNDS)

        def chunk_body(c, carries):
            for r in range(_RPW):
                pltpu.sync_copy(
                    x_hbm.at[base + r, pl.ds(c * _CH, _CH)], ibuf.at[r])

            def vec_body(i, carries):
                new = []
                for r in range(_RPW):
                    v = ibuf[r, pl.ds(i * 16, 16)]
                    s = plsc.cumsum(v) + carries[r]
                    obuf[r, pl.ds(i * 16, 16)] = s
                    new.append(bcast_last(s))
                return tuple(new)

            carries = lax.fori_loop(0, _CH // 16, vec_body, carries)
            for r in range(_RPW):
                pltpu.sync_copy(
                    obuf.at[r], o_hbm.at[base + r, pl.ds(c * _CH, _CH)])
            return carries

        zero = jnp.zeros((16,), jnp.float32)
        lax.fori_loop(0, _NCH, chunk_body, (zero,) * _RPW)

    return run(x)
